# baseline (speedup 1.0000x reference)
import jax
import jax.numpy as jnp
from jax.experimental import pallas as pl
from jax.experimental.pallas import tpu as pltpu

N = 4096
NFEAT = 4096
NCLASS = 8
BLK = 256


def _support_kernel(x_ref, wgc_ref, sup_ref):
    sup_ref[...] = jnp.dot(
        x_ref[...].astype(jnp.bfloat16), wgc_ref[...],
        preferred_element_type=jnp.float32).astype(jnp.bfloat16)


def _agg_kernel(adj_ref, sup_ref, bgc_ref, wlin_ref, blin_ref, ne_ref, y_ref):
    i = pl.program_id(0)
    nb = pl.num_programs(0)

    gc = jnp.dot(adj_ref[...].astype(jnp.bfloat16), sup_ref[...],
                 preferred_element_type=jnp.float32)
    ne = jnp.maximum(gc + bgc_ref[...], 0.0)
    ne_ref[...] = ne
    m = jnp.max(ne, axis=1, keepdims=True)
    ls = ne - m - jnp.log(jnp.sum(jnp.exp(ne - m), axis=1, keepdims=True))
    part = jnp.dot(wlin_ref[...], ls, preferred_element_type=jnp.float32)

    @pl.when(i == 0)
    def _init():
        y_ref[...] = part + blin_ref[...]

    @pl.when(i > 0)
    def _acc():
        y_ref[...] += part


@jax.jit
def kernel(x, adj, W_gc, b_gc, W_lin, b_lin):
    nb = N // BLK
    bgc2 = b_gc.reshape(1, NCLASS)
    blin2 = b_lin.reshape(1, 1)
    wgc16 = W_gc.astype(jnp.bfloat16)

    support = pl.pallas_call(
        _support_kernel,
        grid=(nb,),
        in_specs=[
            pl.BlockSpec((BLK, NFEAT), lambda k: (k, 0)),
            pl.BlockSpec((NFEAT, NCLASS), lambda k: (0, 0)),
        ],
        out_specs=pl.BlockSpec((BLK, NCLASS), lambda k: (k, 0)),
        out_shape=jax.ShapeDtypeStruct((N, NCLASS), jnp.bfloat16),
    )(x, wgc16)

    ne, y = pl.pallas_call(
        _agg_kernel,
        grid=(nb,),
        in_specs=[
            pl.BlockSpec((BLK, N), lambda i: (i, 0)),
            pl.BlockSpec((N, NCLASS), lambda i: (0, 0)),
            pl.BlockSpec((1, NCLASS), lambda i: (0, 0)),
            pl.BlockSpec((1, BLK), lambda i: (0, i)),
            pl.BlockSpec((1, 1), lambda i: (0, 0)),
        ],
        out_specs=[
            pl.BlockSpec((BLK, NCLASS), lambda i: (i, 0)),
            pl.BlockSpec((1, NCLASS), lambda i: (0, 0)),
        ],
        out_shape=[
            jax.ShapeDtypeStruct((N, NCLASS), jnp.float32),
            jax.ShapeDtypeStruct((1, NCLASS), jnp.float32),
        ],
    )(adj, support, bgc2, W_lin, blin2)
    return (y, ne)


# speedup vs baseline: 1.1623x; 1.1623x over previous
import jax
import jax.numpy as jnp
from jax.experimental import pallas as pl
from jax.experimental.pallas import tpu as pltpu

N = 4096
NFEAT = 4096
NCLASS = 8
BLK = 256
HF = NFEAT // 2
HN = N // 2


def _stream_kernel(xa_ref, xb_ref, adja_ref, adjb_ref, wgc_ref, bgc_ref,
                   wlin_ref, blin_ref, ne_ref, y_ref, acc_ref, sup_ref):
    k = pl.program_id(0)
    nb = pl.num_programs(0)

    @pl.when(k < nb - 1)
    def _dot1():
        wgc16 = wgc_ref[...].astype(jnp.bfloat16)
        sup_ref[pl.ds((k % 2) * BLK, BLK), :] = (
            jnp.dot(xa_ref[...].astype(jnp.bfloat16), wgc16[:HF, :],
                    preferred_element_type=jnp.float32)
            + jnp.dot(xb_ref[...].astype(jnp.bfloat16), wgc16[HF:, :],
                      preferred_element_type=jnp.float32)).astype(jnp.bfloat16)

    sup_prev = sup_ref[pl.ds(((k - 1) % 2) * BLK, BLK), :]

    @pl.when(k == 1)
    def _init():
        acc_ref[0:HN, :] = jnp.dot(adja_ref[...].astype(jnp.bfloat16), sup_prev,
                                   preferred_element_type=jnp.float32)
        acc_ref[HN:N, :] = jnp.dot(adjb_ref[...].astype(jnp.bfloat16), sup_prev,
                                   preferred_element_type=jnp.float32)

    @pl.when(k > 1)
    def _acc():
        acc_ref[0:HN, :] += jnp.dot(adja_ref[...].astype(jnp.bfloat16), sup_prev,
                                    preferred_element_type=jnp.float32)
        acc_ref[HN:N, :] += jnp.dot(adjb_ref[...].astype(jnp.bfloat16), sup_prev,
                                    preferred_element_type=jnp.float32)

    @pl.when(k == nb - 1)
    def _writeout():
        ne = jnp.maximum(acc_ref[...] + bgc_ref[...], 0.0)
        ne_ref[...] = ne
        m = jnp.max(ne, axis=1, keepdims=True)
        ls = ne - m - jnp.log(jnp.sum(jnp.exp(ne - m), axis=1, keepdims=True))
        y_ref[...] = jnp.dot(wlin_ref[...], ls,
                             preferred_element_type=jnp.float32) + blin_ref[...]


def _epilogue_kernel(gc_ref, bgc_ref, wlin_ref, blin_ref, ne_ref, y_ref):
    ne = jnp.maximum(gc_ref[...] + bgc_ref[...], 0.0)
    ne_ref[...] = ne
    m = jnp.max(ne, axis=1, keepdims=True)
    ls = ne - m - jnp.log(jnp.sum(jnp.exp(ne - m), axis=1, keepdims=True))
    y_ref[...] = jnp.dot(wlin_ref[...], ls,
                         preferred_element_type=jnp.float32) + blin_ref[...]


@jax.jit
def kernel(x, adj, W_gc, b_gc, W_lin, b_lin):
    nb = NFEAT // BLK
    bgc2 = b_gc.reshape(1, NCLASS)
    blin2 = b_lin.reshape(1, 1)

    ne, y = pl.pallas_call(
        _stream_kernel,
        grid=(nb + 1,),
        in_specs=[
            pl.BlockSpec((BLK, HF), lambda k: (jnp.minimum(k, nb - 1), 0)),
            pl.BlockSpec((BLK, HF), lambda k: (jnp.minimum(k, nb - 1), 1)),
            pl.BlockSpec((HN, BLK), lambda k: (0, jnp.maximum(k - 1, 0))),
            pl.BlockSpec((HN, BLK), lambda k: (1, jnp.maximum(k - 1, 0))),
            pl.BlockSpec((NFEAT, NCLASS), lambda k: (0, 0)),
            pl.BlockSpec((1, NCLASS), lambda k: (0, 0)),
            pl.BlockSpec((1, NFEAT), lambda k: (0, 0)),
            pl.BlockSpec((1, 1), lambda k: (0, 0)),
        ],
        out_specs=[
            pl.BlockSpec((N, NCLASS), lambda k: (0, 0)),
            pl.BlockSpec((1, NCLASS), lambda k: (0, 0)),
        ],
        out_shape=[
            jax.ShapeDtypeStruct((N, NCLASS), jnp.float32),
            jax.ShapeDtypeStruct((1, NCLASS), jnp.float32),
        ],
        scratch_shapes=[
            pltpu.VMEM((N, NCLASS), jnp.float32),
            pltpu.VMEM((2 * BLK, NCLASS), jnp.bfloat16),
        ],
    )(x, x, adj, adj, W_gc, bgc2, W_lin, blin2)
    return (y, ne)
